# baseline (speedup 1.0000x reference)
import numpy as np

import jax
import jax.numpy as jnp
from jax.experimental import pallas as pl
from jax.experimental.pallas import tpu as pltpu

_MASK_RATE = 0.1


def _mask_starts(batch_size: int, T: int):
    len_mask = int(round(T * _MASK_RATE))
    with jax.ensure_compile_time_eval():
        key = jax.random.key(42)
        b = jax.random.randint(key, (batch_size,), 0, T - len_mask)
        starts = [int(v) for v in np.asarray(b)]
    return starts, len_mask


def kernel(input):
    B, C, T, D = input.shape
    starts, len_mask = _mask_starts(B, T)
    s_arr = jnp.asarray(starts, dtype=jnp.int32)

    def body(s_ref, x_ref, o_ref):
        i = pl.program_id(0)
        s = s_ref[i]
        t = jax.lax.broadcasted_iota(jnp.int32, (T, D), 0)
        m = (t >= s) & (t < s + len_mask)
        o_ref[0, 0] = jnp.where(m, jnp.float32(0.0), x_ref[0, 0])

    out = pl.pallas_call(
        body,
        grid=(B, C),
        in_specs=[
            pl.BlockSpec(memory_space=pltpu.SMEM),
            pl.BlockSpec((1, 1, T, D), lambda i, j: (i, j, 0, 0)),
        ],
        out_specs=pl.BlockSpec((1, 1, T, D), lambda i, j: (i, j, 0, 0)),
        out_shape=jax.ShapeDtypeStruct(input.shape, input.dtype),
    )(s_arr, input)
    return out
